# baseline (speedup 1.0000x reference)
import functools

import jax
import jax.numpy as jnp
from jax import lax
from jax.experimental import pallas as pl
from jax.experimental.pallas import tpu as pltpu
from jax.experimental.pallas import tpu_sc as plsc

E_DYN = 8
TOP_K = 2
JITTER_EPS = 0.01
BM = 256
NEG = float("-inf")


def _silu(v):
    return v * jax.nn.sigmoid(v)


def _route_body(x_ref, wr_ref, wv_ref, pp_ref, te_ref):
    xb = x_ref[...]
    logits = jnp.dot(xb, wr_ref[...], preferred_element_type=jnp.float32)
    bt = logits.shape[0]
    col = lax.broadcasted_iota(jnp.int32, (bt, 128), 1)
    dynmask = col < E_DYN
    orig = jnp.where(dynmask, logits, NEG)

    ms = orig
    mults = []
    ams = []
    for _ in range(TOP_K):
        thr = jnp.max(ms, axis=-1, keepdims=True)
        am = jnp.argmax(ms, axis=-1, keepdims=True)
        factor = jnp.maximum(jnp.abs(orig), jnp.abs(thr))
        mask = (thr - orig) / factor > 2.0 * JITTER_EPS
        mg = jnp.where(mask, NEG, ms)
        mgmax = jnp.max(mg, axis=-1, keepdims=True)
        e = jnp.exp(mg - mgmax)
        gates = e / jnp.sum(e, axis=-1, keepdims=True)
        mult = jnp.sum(jnp.where(col == am, gates, 0.0), axis=-1, keepdims=True)
        ms = jnp.where(col == am, NEG, ms)
        mults.append(mult)
        ams.append(am)

    act = (col == ams[0]) | (col == ams[1]) | (col == E_DYN)
    logits9 = jnp.where(col <= E_DYN, logits, NEG)
    gl = jnp.where(act, logits9, NEG)
    glmax = jnp.max(gl, axis=-1, keepdims=True)
    ge = jnp.exp(gl - glmax)
    gw = ge / jnp.sum(ge, axis=-1, keepdims=True)
    gsum = jnp.sum(jnp.where(dynmask, gw, 0.0), axis=-1, keepdims=True)
    gfix = jnp.sum(jnp.where(col == E_DYN, gw, 0.0), axis=-1, keepdims=True)

    w0 = mults[0] * gsum
    w1 = mults[1] * gsum
    wv_ref[...] = (w0 * (col == 0) + w1 * (col == 1) + gfix * (col == 2)
                   ).astype(jnp.float32)

    T = bt
    e0 = ams[0]
    e1 = ams[1]
    oh0 = (col == e0).astype(jnp.float32)
    oh1 = (col == e1).astype(jnp.float32)

    ri = lax.broadcasted_iota(jnp.int32, (T, T), 0)
    ci = lax.broadcasted_iota(jnp.int32, (T, T), 1)
    ltri = (ci <= ri).astype(jnp.float32)
    c0 = jnp.dot(ltri, oh0, preferred_element_type=jnp.float32)
    c1 = jnp.dot(ltri, oh1, preferred_element_type=jnp.float32)

    cnt0 = c0[T - 1:T, :]
    cnt = cnt0 + c1[T - 1:T, :]
    cap = jnp.floor((cnt + (BM - 1)) * (1.0 / BM)) * BM
    ri8 = lax.broadcasted_iota(jnp.int32, (128, 128), 0)
    ci8 = lax.broadcasted_iota(jnp.int32, (128, 128), 1)
    strict = (ri8 < ci8).astype(jnp.float32)
    off_pad = jnp.dot(cap, strict, preferred_element_type=jnp.float32)

    posp0 = jnp.sum(oh0 * (off_pad + c0 - 1.0), axis=-1, keepdims=True)
    posp1 = jnp.sum(oh1 * (off_pad + cnt0 + c1 - 1.0), axis=-1, keepdims=True)
    pp_ref[...] = (posp0 * (col == 0) + posp1 * (col == 1)).astype(jnp.int32)

    jbm = (ri8 * BM).astype(jnp.float32)
    emask = (ci8 >= 1) & (ci8 <= E_DYN)
    offb = jnp.broadcast_to(off_pad, (128, 128))
    ge = ((jbm >= offb) & emask).astype(jnp.float32)
    tile_e = jnp.clip(jnp.sum(ge, axis=-1, keepdims=True), 0, E_DYN - 1)
    total = jnp.sum(jnp.where(ci8 < E_DYN, jnp.broadcast_to(cap, (128, 128)),
                              0.0), axis=-1, keepdims=True)
    tile_v = (jbm < total).astype(jnp.float32)
    te_ref[...] = (tile_e * (ci8 == 0) + tile_v * (ci8 == 1)).astype(jnp.int32)


def _route_call(x, wr_pad):
    T, D = x.shape
    return pl.pallas_call(
        _route_body,
        grid=(1,),
        in_specs=[
            pl.BlockSpec((T, D), lambda m: (0, 0)),
            pl.BlockSpec((D, 128), lambda m: (0, 0)),
        ],
        out_specs=[
            pl.BlockSpec((T, 128), lambda m: (0, 0)),
            pl.BlockSpec((T, 128), lambda m: (0, 0)),
            pl.BlockSpec((128, 128), lambda m: (0, 0)),
        ],
        out_shape=[
            jax.ShapeDtypeStruct((T, 128), jnp.float32),
            jax.ShapeDtypeStruct((T, 128), jnp.int32),
            jax.ShapeDtypeStruct((128, 128), jnp.int32),
        ],
        compiler_params=pltpu.CompilerParams(
            dimension_semantics=("arbitrary",)),
    )(x, wr_pad)


_NBUF = 3


@functools.lru_cache(maxsize=None)
def _make_sc_gather(N, D, NW, C, CH, NC, dtype=jnp.float32):
    B = NW * C * CH
    mesh = plsc.VectorSubcoreMesh(core_axis_name="c", subcore_axis_name="s")

    @functools.partial(
        pl.kernel, mesh=mesh,
        out_type=jax.ShapeDtypeStruct((B, D), dtype),
        scratch_types=[
            pltpu.VMEM((C, CH), jnp.int32),
        ] + [pltpu.VMEM((CH, D), dtype) for _ in range(_NBUF)]
          + [pltpu.SemaphoreType.DMA for _ in range(2 * _NBUF)],
    )
    def k(table_hbm, idx_hbm, out_hbm, idx_v, *rest):
        bufs = rest[:_NBUF]
        gsem = rest[_NBUF:2 * _NBUF]
        ssem = rest[2 * _NBUF:]
        wid = lax.axis_index("s") * NC + lax.axis_index("c")
        base = wid * (C * CH)
        pltpu.sync_copy(idx_hbm.at[wid], idx_v)

        def gather(c):
            return pltpu.async_copy(table_hbm.at[idx_v.at[c]],
                                    bufs[c % _NBUF], gsem[c % _NBUF])

        def store(c):
            return pltpu.async_copy(bufs[c % _NBUF],
                                    out_hbm.at[pl.ds(base + c * CH, CH)],
                                    ssem[c % _NBUF])

        g = {}
        s = {}
        for c in range(min(_NBUF, C)):
            g[c] = gather(c)
        for c in range(C):
            g[c].wait()
            s[c] = store(c)
            nxt = c + _NBUF
            if nxt < C:
                s[c].wait()
                g[nxt] = gather(nxt)
        for c in range(max(0, C - _NBUF), C):
            s[c].wait()

    return k


@functools.lru_cache(maxsize=None)
def _make_sc_scatter(N, D, P, NW, C, CH, NC):
    mesh = plsc.VectorSubcoreMesh(core_axis_name="c", subcore_axis_name="s")

    @functools.partial(
        pl.kernel, mesh=mesh,
        out_type=jax.ShapeDtypeStruct((P, D), jnp.float32),
        scratch_types=[
            pltpu.VMEM((C, CH), jnp.int32),
        ] + [pltpu.VMEM((CH, D), jnp.float32) for _ in range(_NBUF)]
          + [pltpu.SemaphoreType.DMA for _ in range(2 * _NBUF)],
    )
    def k(src_hbm, idx_hbm, out_hbm, idx_v, *rest):
        bufs = rest[:_NBUF]
        gsem = rest[_NBUF:2 * _NBUF]
        ssem = rest[2 * _NBUF:]
        wid = lax.axis_index("s") * NC + lax.axis_index("c")
        base = lax.rem(wid * (C * CH), N)
        pltpu.sync_copy(idx_hbm.at[wid], idx_v)

        def load(c):
            return pltpu.async_copy(src_hbm.at[pl.ds(base + c * CH, CH)],
                                    bufs[c % _NBUF], gsem[c % _NBUF])

        def scat(c):
            return pltpu.async_copy(bufs[c % _NBUF],
                                    out_hbm.at[idx_v.at[c]],
                                    ssem[c % _NBUF])

        g = {}
        s = {}
        for c in range(min(_NBUF, C)):
            g[c] = load(c)
        for c in range(C):
            g[c].wait()
            s[c] = scat(c)
            nxt = c + _NBUF
            if nxt < C:
                s[c].wait()
                g[nxt] = load(nxt)
        for c in range(max(0, C - _NBUF), C):
            s[c].wait()

    return k


def _group_body(te_ref, tv_ref, xp_ref, wg_ref, wu_ref, wd_ref, yp_ref):
    @pl.when(tv_ref[pl.program_id(0)] != 0)
    def _():
        xb = xp_ref[...].astype(jnp.bfloat16)
        hg = _silu(jnp.dot(xb, wg_ref[0], preferred_element_type=jnp.float32))
        hu = jnp.dot(xb, wu_ref[0], preferred_element_type=jnp.float32)
        h = (hg * hu).astype(jnp.bfloat16)
        yp_ref[...] = jnp.dot(h, wd_ref[0], preferred_element_type=jnp.float32)


def _group_call(tile_e, tile_v, xp, wg, wu, wd):
    P, D = xp.shape
    F = wg.shape[-1]
    ntiles = P // BM
    grid_spec = pltpu.PrefetchScalarGridSpec(
        num_scalar_prefetch=2,
        grid=(ntiles,),
        in_specs=[
            pl.BlockSpec((BM, D), lambda j, te, tv: (j, 0)),
            pl.BlockSpec((1, D, F), lambda j, te, tv: (te[j], 0, 0)),
            pl.BlockSpec((1, D, F), lambda j, te, tv: (te[j], 0, 0)),
            pl.BlockSpec((1, F, D), lambda j, te, tv: (te[j], 0, 0)),
        ],
        out_specs=pl.BlockSpec((BM, D), lambda j, te, tv: (j, 0)),
    )
    return pl.pallas_call(
        _group_body,
        grid_spec=grid_spec,
        out_shape=jax.ShapeDtypeStruct((P, D), jnp.float32),
        compiler_params=pltpu.CompilerParams(
            dimension_semantics=("arbitrary",)),
    )(tile_e, tile_v, xp, wg, wu, wd)


def _combine_body(y0_ref, y1_ref, x_ref, wg_ref, wu_ref, wd_ref, wv_ref,
                  out_ref):
    w0 = wv_ref[:, 0:1]
    w1 = wv_ref[:, 1:2]
    gfix = wv_ref[:, 2:3]
    xb = x_ref[...].astype(jnp.bfloat16)
    hg = _silu(jnp.dot(xb, wg_ref[...], preferred_element_type=jnp.float32))
    hu = jnp.dot(xb, wu_ref[...], preferred_element_type=jnp.float32)
    h = (hg * hu).astype(jnp.bfloat16)
    ysh = jnp.dot(h, wd_ref[...], preferred_element_type=jnp.float32) * gfix
    out_ref[...] = y0_ref[...] * w0 + y1_ref[...] * w1 + ysh


def _combine_call(ygath, x, wg_sh, wu_sh, wd_sh, wv):
    TK, D = ygath.shape
    T = TK // TOP_K
    BT = 512
    nb = T // BT
    return pl.pallas_call(
        _combine_body,
        grid=(nb,),
        in_specs=[
            pl.BlockSpec((BT, D), lambda m: (m, 0)),
            pl.BlockSpec((BT, D), lambda m, _nb=nb: (m + _nb, 0)),
            pl.BlockSpec((BT, D), lambda m: (m, 0)),
            pl.BlockSpec(wg_sh.shape, lambda m: (0, 0)),
            pl.BlockSpec(wu_sh.shape, lambda m: (0, 0)),
            pl.BlockSpec(wd_sh.shape, lambda m: (0, 0)),
            pl.BlockSpec((BT, 128), lambda m: (m, 0)),
        ],
        out_specs=pl.BlockSpec((BT, D), lambda m: (m, 0)),
        out_shape=jax.ShapeDtypeStruct((T, D), jnp.float32),
        compiler_params=pltpu.CompilerParams(
            dimension_semantics=("arbitrary",)),
    )(ygath, ygath, x, wg_sh, wu_sh, wd_sh, wv)


@jax.jit
def _run(hidden_states, W_router, Wg_dyn, Wu_dyn, Wd_dyn, Wg_sh, Wu_sh, Wd_sh):
    B, S, D = hidden_states.shape
    x = hidden_states.reshape(-1, D)
    T = x.shape[0]
    NSLOT = TOP_K * T
    P = NSLOT + E_DYN * BM

    wr_pad = jnp.zeros((D, 128), jnp.float32).at[:, :E_DYN + 1].set(W_router)
    wv, pp, te2 = _route_call(x, wr_pad)

    posp = jnp.concatenate([pp[:, 0], pp[:, 1]])
    tile_e = te2[:P // BM, 0]
    tile_v = te2[:P // BM, 1]

    xp = _sc_scatter(x, posp.reshape(32, -1, 16), P)
    yp = _group_call(tile_e, tile_v, xp,
                     Wg_dyn.astype(jnp.bfloat16), Wu_dyn.astype(jnp.bfloat16),
                     Wd_dyn.astype(jnp.bfloat16))
    ygath = _sc_gather(yp, posp.reshape(32, -1, 16))
    out = _combine_call(ygath, x, Wg_sh[0].astype(jnp.bfloat16),
                        Wu_sh[0].astype(jnp.bfloat16),
                        Wd_sh[0].astype(jnp.bfloat16), wv)
    return out.reshape(B, S, D)


def _sc_gather(table, idx3):
    fn = _make_sc_gather(table.shape[0], table.shape[1],
                         idx3.shape[0], idx3.shape[1], idx3.shape[2], 2,
                         table.dtype.type)
    return fn(table, idx3)


def _sc_scatter(src, idx3, P):
    fn = _make_sc_scatter(src.shape[0], src.shape[1], P,
                          idx3.shape[0], idx3.shape[1], idx3.shape[2], 2)
    return fn(src, idx3)


def kernel(hidden_states, W_router, Wg_dyn, Wu_dyn, Wd_dyn, Wg_sh, Wu_sh, Wd_sh):
    return _run(hidden_states, W_router, Wg_dyn, Wu_dyn, Wd_dyn,
                Wg_sh, Wu_sh, Wd_sh)


# speedup vs baseline: 1.1461x; 1.1461x over previous
import functools

import jax
import jax.numpy as jnp
from jax import lax
from jax.experimental import pallas as pl
from jax.experimental.pallas import tpu as pltpu
from jax.experimental.pallas import tpu_sc as plsc

E_DYN = 8
TOP_K = 2
JITTER_EPS = 0.01
BM = 256
NEG = float("-inf")


def _silu(v):
    return v * jax.nn.sigmoid(v)


def _route_body(x_ref, wr_ref, wv_ref, pp_ref, te_ref):
    xb = x_ref[...]
    logits = jnp.dot(xb, wr_ref[...], preferred_element_type=jnp.float32)
    bt = logits.shape[0]
    col = lax.broadcasted_iota(jnp.int32, (bt, 128), 1)
    dynmask = col < E_DYN
    orig = jnp.where(dynmask, logits, NEG)

    ms = orig
    mults = []
    ams = []
    for _ in range(TOP_K):
        thr = jnp.max(ms, axis=-1, keepdims=True)
        am = jnp.argmax(ms, axis=-1, keepdims=True)
        factor = jnp.maximum(jnp.abs(orig), jnp.abs(thr))
        mask = (thr - orig) / factor > 2.0 * JITTER_EPS
        mg = jnp.where(mask, NEG, ms)
        mgmax = jnp.max(mg, axis=-1, keepdims=True)
        e = jnp.exp(mg - mgmax)
        gates = e / jnp.sum(e, axis=-1, keepdims=True)
        mult = jnp.sum(jnp.where(col == am, gates, 0.0), axis=-1, keepdims=True)
        ms = jnp.where(col == am, NEG, ms)
        mults.append(mult)
        ams.append(am)

    act = (col == ams[0]) | (col == ams[1]) | (col == E_DYN)
    logits9 = jnp.where(col <= E_DYN, logits, NEG)
    gl = jnp.where(act, logits9, NEG)
    glmax = jnp.max(gl, axis=-1, keepdims=True)
    ge = jnp.exp(gl - glmax)
    gw = ge / jnp.sum(ge, axis=-1, keepdims=True)
    gsum = jnp.sum(jnp.where(dynmask, gw, 0.0), axis=-1, keepdims=True)
    gfix = jnp.sum(jnp.where(col == E_DYN, gw, 0.0), axis=-1, keepdims=True)

    w0 = mults[0] * gsum
    w1 = mults[1] * gsum
    wv_ref[...] = (w0 * (col == 0) + w1 * (col == 1) + gfix * (col == 2)
                   ).astype(jnp.float32)

    T = bt
    e0 = ams[0]
    e1 = ams[1]
    oh0 = (col == e0).astype(jnp.float32)
    oh1 = (col == e1).astype(jnp.float32)

    ri = lax.broadcasted_iota(jnp.int32, (T, T), 0)
    ci = lax.broadcasted_iota(jnp.int32, (T, T), 1)
    ltri = (ci <= ri).astype(jnp.float32)
    c0 = jnp.dot(ltri, oh0, preferred_element_type=jnp.float32)
    c1 = jnp.dot(ltri, oh1, preferred_element_type=jnp.float32)

    cnt0 = c0[T - 1:T, :]
    cnt = cnt0 + c1[T - 1:T, :]
    cap = jnp.floor((cnt + (BM - 1)) * (1.0 / BM)) * BM
    ri8 = lax.broadcasted_iota(jnp.int32, (128, 128), 0)
    ci8 = lax.broadcasted_iota(jnp.int32, (128, 128), 1)
    strict = (ri8 < ci8).astype(jnp.float32)
    off_pad = jnp.dot(cap, strict, preferred_element_type=jnp.float32)

    posp0 = jnp.sum(oh0 * (off_pad + c0 - 1.0), axis=-1, keepdims=True)
    posp1 = jnp.sum(oh1 * (off_pad + cnt0 + c1 - 1.0), axis=-1, keepdims=True)
    pp_ref[...] = (posp0 * (col == 0) + posp1 * (col == 1)).astype(jnp.int32)

    jbm = (ri8 * BM).astype(jnp.float32)
    emask = (ci8 >= 1) & (ci8 <= E_DYN)
    offb = jnp.broadcast_to(off_pad, (128, 128))
    ge = ((jbm >= offb) & emask).astype(jnp.float32)
    tile_e = jnp.clip(jnp.sum(ge, axis=-1, keepdims=True), 0, E_DYN - 1)
    total = jnp.sum(jnp.where(ci8 < E_DYN, jnp.broadcast_to(cap, (128, 128)),
                              0.0), axis=-1, keepdims=True)
    tile_v = (jbm < total).astype(jnp.float32)
    te_ref[...] = (tile_e * (ci8 == 0) + tile_v * (ci8 == 1)).astype(jnp.int32)


def _route_call(x, wr_pad):
    T, D = x.shape
    return pl.pallas_call(
        _route_body,
        grid=(1,),
        in_specs=[
            pl.BlockSpec((T, D), lambda m: (0, 0)),
            pl.BlockSpec((D, 128), lambda m: (0, 0)),
        ],
        out_specs=[
            pl.BlockSpec((T, 128), lambda m: (0, 0)),
            pl.BlockSpec((T, 128), lambda m: (0, 0)),
            pl.BlockSpec((128, 128), lambda m: (0, 0)),
        ],
        out_shape=[
            jax.ShapeDtypeStruct((T, 128), jnp.float32),
            jax.ShapeDtypeStruct((T, 128), jnp.int32),
            jax.ShapeDtypeStruct((128, 128), jnp.int32),
        ],
        compiler_params=pltpu.CompilerParams(
            dimension_semantics=("arbitrary",)),
    )(x, wr_pad)


_NBUF = 3


@functools.lru_cache(maxsize=None)
def _make_sc_gather(N, D, NW, C, CH, NC, dtype=jnp.float32):
    B = NW * C * CH
    mesh = plsc.VectorSubcoreMesh(core_axis_name="c", subcore_axis_name="s")

    @functools.partial(
        pl.kernel, mesh=mesh,
        out_type=jax.ShapeDtypeStruct((B, D), dtype),
        scratch_types=[
            pltpu.VMEM((C, CH), jnp.int32),
        ] + [pltpu.VMEM((CH, D), dtype) for _ in range(_NBUF)]
          + [pltpu.SemaphoreType.DMA for _ in range(2 * _NBUF)],
    )
    def k(table_hbm, idx_hbm, out_hbm, idx_v, *rest):
        bufs = rest[:_NBUF]
        gsem = rest[_NBUF:2 * _NBUF]
        ssem = rest[2 * _NBUF:]
        wid = lax.axis_index("s") * NC + lax.axis_index("c")
        base = wid * (C * CH)
        pltpu.sync_copy(idx_hbm.at[wid], idx_v)

        def gather(c):
            return pltpu.async_copy(table_hbm.at[idx_v.at[c]],
                                    bufs[c % _NBUF], gsem[c % _NBUF])

        def store(c):
            return pltpu.async_copy(bufs[c % _NBUF],
                                    out_hbm.at[pl.ds(base + c * CH, CH)],
                                    ssem[c % _NBUF])

        g = {}
        s = {}
        for c in range(min(_NBUF, C)):
            g[c] = gather(c)
        for c in range(C):
            g[c].wait()
            s[c] = store(c)
            nxt = c + _NBUF
            if nxt < C:
                s[c].wait()
                g[nxt] = gather(nxt)
        for c in range(max(0, C - _NBUF), C):
            s[c].wait()

    return k


@functools.lru_cache(maxsize=None)
def _make_sc_scatter(N, D, P, NW, C, CH, NC):
    mesh = plsc.VectorSubcoreMesh(core_axis_name="c", subcore_axis_name="s")

    @functools.partial(
        pl.kernel, mesh=mesh,
        out_type=jax.ShapeDtypeStruct((P, D), jnp.float32),
        scratch_types=[
            pltpu.VMEM((C, CH), jnp.int32),
        ] + [pltpu.VMEM((CH, D), jnp.float32) for _ in range(_NBUF)]
          + [pltpu.SemaphoreType.DMA for _ in range(2 * _NBUF)],
    )
    def k(src_hbm, idx_hbm, out_hbm, idx_v, *rest):
        bufs = rest[:_NBUF]
        gsem = rest[_NBUF:2 * _NBUF]
        ssem = rest[2 * _NBUF:]
        wid = lax.axis_index("s") * NC + lax.axis_index("c")
        base = lax.rem(wid * (C * CH), N)
        pltpu.sync_copy(idx_hbm.at[wid], idx_v)

        def load(c):
            return pltpu.async_copy(src_hbm.at[pl.ds(base + c * CH, CH)],
                                    bufs[c % _NBUF], gsem[c % _NBUF])

        def scat(c):
            return pltpu.async_copy(bufs[c % _NBUF],
                                    out_hbm.at[idx_v.at[c]],
                                    ssem[c % _NBUF])

        g = {}
        s = {}
        for c in range(min(_NBUF, C)):
            g[c] = load(c)
        for c in range(C):
            g[c].wait()
            s[c] = scat(c)
            nxt = c + _NBUF
            if nxt < C:
                s[c].wait()
                g[nxt] = load(nxt)
        for c in range(max(0, C - _NBUF), C):
            s[c].wait()

    return k


def _group_body(te_ref, tv_ref, xp_ref, wg_ref, wu_ref, wd_ref, yp_ref):
    @pl.when(tv_ref[pl.program_id(0)] != 0)
    def _():
        xb = xp_ref[...].astype(jnp.bfloat16)
        wg = wg_ref[0].astype(jnp.bfloat16)
        wu = wu_ref[0].astype(jnp.bfloat16)
        hg = _silu(jnp.dot(xb, wg, preferred_element_type=jnp.float32))
        hu = jnp.dot(xb, wu, preferred_element_type=jnp.float32)
        h = (hg * hu).astype(jnp.bfloat16)
        yp_ref[...] = jnp.dot(h, wd_ref[0].astype(jnp.bfloat16),
                              preferred_element_type=jnp.float32)


def _group_call(tile_e, tile_v, xp, wg, wu, wd):
    P, D = xp.shape
    F = wg.shape[-1]
    ntiles = P // BM
    grid_spec = pltpu.PrefetchScalarGridSpec(
        num_scalar_prefetch=2,
        grid=(ntiles,),
        in_specs=[
            pl.BlockSpec((BM, D), lambda j, te, tv: (j, 0)),
            pl.BlockSpec((1, D, F), lambda j, te, tv: (te[j], 0, 0)),
            pl.BlockSpec((1, D, F), lambda j, te, tv: (te[j], 0, 0)),
            pl.BlockSpec((1, F, D), lambda j, te, tv: (te[j], 0, 0)),
        ],
        out_specs=pl.BlockSpec((BM, D), lambda j, te, tv: (j, 0)),
    )
    return pl.pallas_call(
        _group_body,
        grid_spec=grid_spec,
        out_shape=jax.ShapeDtypeStruct((P, D), jnp.float32),
        compiler_params=pltpu.CompilerParams(
            dimension_semantics=("arbitrary",)),
    )(tile_e, tile_v, xp, wg, wu, wd)


def _combine_body(y0_ref, y1_ref, x_ref, wg_ref, wu_ref, wd_ref, wv_ref,
                  *rest):
    out_ref = rest[-1]
    w0 = wv_ref[:, 0:1]
    w1 = wv_ref[:, 1:2]
    gfix = wv_ref[:, 2:3]
    xb = x_ref[...].astype(jnp.bfloat16)
    hg = _silu(jnp.dot(xb, wg_ref[...].astype(jnp.bfloat16),
                       preferred_element_type=jnp.float32))
    hu = jnp.dot(xb, wu_ref[...].astype(jnp.bfloat16),
                 preferred_element_type=jnp.float32)
    h = (hg * hu).astype(jnp.bfloat16)
    ysh = jnp.dot(h, wd_ref[...].astype(jnp.bfloat16),
                  preferred_element_type=jnp.float32) * gfix
    out_ref[...] = y0_ref[...] * w0 + y1_ref[...] * w1 + ysh


def _combine_call(ygath, x, wg_sh, wu_sh, wd_sh, wv, half, T, prev=None):
    TK, D = ygath.shape
    TH = TK // TOP_K
    BT = 512
    nb = TH // BT
    off = half * nb
    in_specs = [
        pl.BlockSpec((BT, D), lambda m: (m, 0)),
        pl.BlockSpec((BT, D), lambda m, _nb=nb: (m + _nb, 0)),
        pl.BlockSpec((BT, D), lambda m, _o=off: (m + _o, 0)),
        pl.BlockSpec(wg_sh.shape, lambda m: (0, 0)),
        pl.BlockSpec(wu_sh.shape, lambda m: (0, 0)),
        pl.BlockSpec(wd_sh.shape, lambda m: (0, 0)),
        pl.BlockSpec((BT, 128), lambda m, _o=off: (m + _o, 0)),
    ]
    args = [ygath, ygath, x, wg_sh, wu_sh, wd_sh, wv]
    aliases = {}
    if prev is not None:
        in_specs.append(pl.BlockSpec(memory_space=pl.ANY))
        args.append(prev)
        aliases = {7: 0}
    return pl.pallas_call(
        _combine_body,
        grid=(nb,),
        in_specs=in_specs,
        out_specs=pl.BlockSpec((BT, D), lambda m, _o=off: (m + _o, 0)),
        out_shape=jax.ShapeDtypeStruct((T, D), jnp.float32),
        input_output_aliases=aliases,
        compiler_params=pltpu.CompilerParams(
            dimension_semantics=("arbitrary",)),
    )(*args)


@jax.jit
def _run(hidden_states, W_router, Wg_dyn, Wu_dyn, Wd_dyn, Wg_sh, Wu_sh, Wd_sh):
    B, S, D = hidden_states.shape
    x = hidden_states.reshape(-1, D)
    T = x.shape[0]
    NSLOT = TOP_K * T
    P = NSLOT + E_DYN * BM

    wr_pad = jnp.zeros((D, 128), jnp.float32).at[:, :E_DYN + 1].set(W_router)
    wv, pp, te2 = _route_call(x, wr_pad)

    posp = jnp.concatenate([pp[:, 0], pp[:, 1]])
    tile_e = te2[:P // BM, 0]
    tile_v = te2[:P // BM, 1]

    xp = _sc_scatter(x, posp.reshape(32, -1, 16), P)
    yp = _group_call(tile_e, tile_v, xp, Wg_dyn, Wu_dyn, Wd_dyn)

    TH = T // 2
    p0 = pp[:, 0]
    p1 = pp[:, 1]
    out = None
    for h in range(2):
        idxh = jnp.concatenate([p0[h * TH:(h + 1) * TH],
                                p1[h * TH:(h + 1) * TH]]).reshape(32, -1, 16)
        ygh = _sc_gather(yp, idxh)
        out = _combine_call(ygh, x, Wg_sh[0], Wu_sh[0], Wd_sh[0], wv,
                            h, T, prev=out)
    return out.reshape(B, S, D)


def _sc_gather(table, idx3):
    fn = _make_sc_gather(table.shape[0], table.shape[1],
                         idx3.shape[0], idx3.shape[1], idx3.shape[2], 2,
                         table.dtype.type)
    return fn(table, idx3)


def _sc_scatter(src, idx3, P):
    fn = _make_sc_scatter(src.shape[0], src.shape[1], P,
                          idx3.shape[0], idx3.shape[1], idx3.shape[2], 2)
    return fn(src, idx3)


def kernel(hidden_states, W_router, Wg_dyn, Wu_dyn, Wd_dyn, Wg_sh, Wu_sh, Wd_sh):
    return _run(hidden_states, W_router, Wg_dyn, Wu_dyn, Wd_dyn,
                Wg_sh, Wu_sh, Wd_sh)


# speedup vs baseline: 1.1654x; 1.0168x over previous
import functools

import jax
import jax.numpy as jnp
from jax import lax
from jax.experimental import pallas as pl
from jax.experimental.pallas import tpu as pltpu
from jax.experimental.pallas import tpu_sc as plsc

E_DYN = 8
TOP_K = 2
JITTER_EPS = 0.01
BM = 256
NEG = float("-inf")


def _silu(v):
    return v * jax.nn.sigmoid(v)


def _route_body(x_ref, wr_ref, wv_ref, pp_ref, te_ref):
    xb = x_ref[...]
    logits = jnp.dot(xb, wr_ref[...], preferred_element_type=jnp.float32)
    bt = logits.shape[0]
    col = lax.broadcasted_iota(jnp.int32, (bt, 128), 1)
    dynmask = col < E_DYN
    orig = jnp.where(dynmask, logits, NEG)

    ms = orig
    mults = []
    ams = []
    for _ in range(TOP_K):
        thr = jnp.max(ms, axis=-1, keepdims=True)
        am = jnp.argmax(ms, axis=-1, keepdims=True)
        factor = jnp.maximum(jnp.abs(orig), jnp.abs(thr))
        mask = (thr - orig) / factor > 2.0 * JITTER_EPS
        mg = jnp.where(mask, NEG, ms)
        mgmax = jnp.max(mg, axis=-1, keepdims=True)
        e = jnp.exp(mg - mgmax)
        gates = e / jnp.sum(e, axis=-1, keepdims=True)
        mult = jnp.sum(jnp.where(col == am, gates, 0.0), axis=-1, keepdims=True)
        ms = jnp.where(col == am, NEG, ms)
        mults.append(mult)
        ams.append(am)

    act = (col == ams[0]) | (col == ams[1]) | (col == E_DYN)
    logits9 = jnp.where(col <= E_DYN, logits, NEG)
    gl = jnp.where(act, logits9, NEG)
    glmax = jnp.max(gl, axis=-1, keepdims=True)
    ge = jnp.exp(gl - glmax)
    gw = ge / jnp.sum(ge, axis=-1, keepdims=True)
    gsum = jnp.sum(jnp.where(dynmask, gw, 0.0), axis=-1, keepdims=True)
    gfix = jnp.sum(jnp.where(col == E_DYN, gw, 0.0), axis=-1, keepdims=True)

    w0 = mults[0] * gsum
    w1 = mults[1] * gsum
    wv_ref[...] = (w0 * (col == 0) + w1 * (col == 1) + gfix * (col == 2)
                   ).astype(jnp.float32)

    T = bt
    e0 = ams[0]
    e1 = ams[1]
    oh0 = (col == e0).astype(jnp.float32)
    oh1 = (col == e1).astype(jnp.float32)

    ri = lax.broadcasted_iota(jnp.int32, (T, T), 0)
    ci = lax.broadcasted_iota(jnp.int32, (T, T), 1)
    ltri = (ci <= ri).astype(jnp.float32)
    c0 = jnp.dot(ltri, oh0, preferred_element_type=jnp.float32)
    c1 = jnp.dot(ltri, oh1, preferred_element_type=jnp.float32)

    cnt0 = c0[T - 1:T, :]
    cnt = cnt0 + c1[T - 1:T, :]
    cap = jnp.floor((cnt + (BM - 1)) * (1.0 / BM)) * BM
    ri8 = lax.broadcasted_iota(jnp.int32, (128, 128), 0)
    ci8 = lax.broadcasted_iota(jnp.int32, (128, 128), 1)
    strict = (ri8 < ci8).astype(jnp.float32)
    off_pad = jnp.dot(cap, strict, preferred_element_type=jnp.float32)

    posp0 = jnp.sum(oh0 * (off_pad + c0 - 1.0), axis=-1, keepdims=True)
    posp1 = jnp.sum(oh1 * (off_pad + cnt0 + c1 - 1.0), axis=-1, keepdims=True)
    pp_ref[...] = (posp0 * (col == 0) + posp1 * (col == 1)).astype(jnp.int32)

    jbm = (ri8 * BM).astype(jnp.float32)
    emask = (ci8 >= 1) & (ci8 <= E_DYN)
    offb = jnp.broadcast_to(off_pad, (128, 128))
    ge = ((jbm >= offb) & emask).astype(jnp.float32)
    tile_e = jnp.clip(jnp.sum(ge, axis=-1, keepdims=True), 0, E_DYN - 1)
    total = jnp.sum(jnp.where(ci8 < E_DYN, jnp.broadcast_to(cap, (128, 128)),
                              0.0), axis=-1, keepdims=True)
    tile_v = (jbm < total).astype(jnp.float32)
    te_ref[...] = (tile_e * (ci8 == 0) + tile_v * (ci8 == 1)).astype(jnp.int32)


def _route_call(x, wr_pad):
    T, D = x.shape
    return pl.pallas_call(
        _route_body,
        grid=(1,),
        in_specs=[
            pl.BlockSpec((T, D), lambda m: (0, 0)),
            pl.BlockSpec((D, 128), lambda m: (0, 0)),
        ],
        out_specs=[
            pl.BlockSpec((T, 128), lambda m: (0, 0)),
            pl.BlockSpec((T, 128), lambda m: (0, 0)),
            pl.BlockSpec((128, 128), lambda m: (0, 0)),
        ],
        out_shape=[
            jax.ShapeDtypeStruct((T, 128), jnp.float32),
            jax.ShapeDtypeStruct((T, 128), jnp.int32),
            jax.ShapeDtypeStruct((128, 128), jnp.int32),
        ],
        compiler_params=pltpu.CompilerParams(
            dimension_semantics=("arbitrary",)),
    )(x, wr_pad)


_NBUF = 3


@functools.lru_cache(maxsize=None)
def _make_sc_gather(N, D, NW, C, CH, NC, dtype=jnp.float32):
    B = NW * C * CH
    mesh = plsc.VectorSubcoreMesh(core_axis_name="c", subcore_axis_name="s")

    @functools.partial(
        pl.kernel, mesh=mesh,
        out_type=jax.ShapeDtypeStruct((B, D), dtype),
        scratch_types=[
            pltpu.VMEM((C, CH), jnp.int32),
        ] + [pltpu.VMEM((CH, D), dtype) for _ in range(_NBUF)]
          + [pltpu.SemaphoreType.DMA for _ in range(2 * _NBUF)],
    )
    def k(table_hbm, idx_hbm, out_hbm, idx_v, *rest):
        bufs = rest[:_NBUF]
        gsem = rest[_NBUF:2 * _NBUF]
        ssem = rest[2 * _NBUF:]
        wid = lax.axis_index("s") * NC + lax.axis_index("c")
        base = wid * (C * CH)
        pltpu.sync_copy(idx_hbm.at[wid], idx_v)

        def gather(c):
            return pltpu.async_copy(table_hbm.at[idx_v.at[c]],
                                    bufs[c % _NBUF], gsem[c % _NBUF])

        def store(c):
            return pltpu.async_copy(bufs[c % _NBUF],
                                    out_hbm.at[pl.ds(base + c * CH, CH)],
                                    ssem[c % _NBUF])

        g = {}
        s = {}
        for c in range(min(_NBUF, C)):
            g[c] = gather(c)
        for c in range(C):
            g[c].wait()
            s[c] = store(c)
            nxt = c + _NBUF
            if nxt < C:
                s[c].wait()
                g[nxt] = gather(nxt)
        for c in range(max(0, C - _NBUF), C):
            s[c].wait()

    return k


@functools.lru_cache(maxsize=None)
def _make_sc_scatter(N, D, P, NW, C, CH, NC):
    mesh = plsc.VectorSubcoreMesh(core_axis_name="c", subcore_axis_name="s")

    @functools.partial(
        pl.kernel, mesh=mesh,
        out_type=jax.ShapeDtypeStruct((P, D), jnp.float32),
        scratch_types=[
            pltpu.VMEM((C, CH), jnp.int32),
        ] + [pltpu.VMEM((CH, D), jnp.float32) for _ in range(_NBUF)]
          + [pltpu.SemaphoreType.DMA for _ in range(2 * _NBUF)],
    )
    def k(src_hbm, idx_hbm, out_hbm, idx_v, *rest):
        bufs = rest[:_NBUF]
        gsem = rest[_NBUF:2 * _NBUF]
        ssem = rest[2 * _NBUF:]
        wid = lax.axis_index("s") * NC + lax.axis_index("c")
        base = lax.rem(wid * (C * CH), N)
        pltpu.sync_copy(idx_hbm.at[wid], idx_v)

        def load(c):
            return pltpu.async_copy(src_hbm.at[pl.ds(base + c * CH, CH)],
                                    bufs[c % _NBUF], gsem[c % _NBUF])

        def scat(c):
            return pltpu.async_copy(bufs[c % _NBUF],
                                    out_hbm.at[idx_v.at[c]],
                                    ssem[c % _NBUF])

        g = {}
        s = {}
        for c in range(min(_NBUF, C)):
            g[c] = load(c)
        for c in range(C):
            g[c].wait()
            s[c] = scat(c)
            nxt = c + _NBUF
            if nxt < C:
                s[c].wait()
                g[nxt] = load(nxt)
        for c in range(max(0, C - _NBUF), C):
            s[c].wait()

    return k


def _group_body(te_ref, tv_ref, xp_ref, wg_ref, wu_ref, wd_ref, yp_ref):
    @pl.when(tv_ref[pl.program_id(0)] != 0)
    def _():
        xb = xp_ref[...].astype(jnp.bfloat16)
        wg = wg_ref[0].astype(jnp.bfloat16)
        wu = wu_ref[0].astype(jnp.bfloat16)
        hg = _silu(jnp.dot(xb, wg, preferred_element_type=jnp.float32))
        hu = jnp.dot(xb, wu, preferred_element_type=jnp.float32)
        h = (hg * hu).astype(jnp.bfloat16)
        yp_ref[...] = jnp.dot(h, wd_ref[0].astype(jnp.bfloat16),
                              preferred_element_type=jnp.float32)


def _group_call(tile_e, tile_v, xp, wg, wu, wd):
    P, D = xp.shape
    F = wg.shape[-1]
    ntiles = P // BM
    grid_spec = pltpu.PrefetchScalarGridSpec(
        num_scalar_prefetch=2,
        grid=(ntiles,),
        in_specs=[
            pl.BlockSpec((BM, D), lambda j, te, tv: (j, 0)),
            pl.BlockSpec((1, D, F), lambda j, te, tv: (te[j], 0, 0)),
            pl.BlockSpec((1, D, F), lambda j, te, tv: (te[j], 0, 0)),
            pl.BlockSpec((1, F, D), lambda j, te, tv: (te[j], 0, 0)),
        ],
        out_specs=pl.BlockSpec((BM, D), lambda j, te, tv: (j, 0)),
    )
    return pl.pallas_call(
        _group_body,
        grid_spec=grid_spec,
        out_shape=jax.ShapeDtypeStruct((P, D), jnp.float32),
        compiler_params=pltpu.CompilerParams(
            dimension_semantics=("arbitrary",)),
    )(tile_e, tile_v, xp, wg, wu, wd)


def _combine_body(y0_ref, y1_ref, x_ref, wg_ref, wu_ref, wd_ref, wv_ref,
                  *rest):
    out_ref = rest[-1]
    w0 = wv_ref[:, 0:1]
    w1 = wv_ref[:, 1:2]
    gfix = wv_ref[:, 2:3]
    xb = x_ref[...].astype(jnp.bfloat16)
    hg = _silu(jnp.dot(xb, wg_ref[...].astype(jnp.bfloat16),
                       preferred_element_type=jnp.float32))
    hu = jnp.dot(xb, wu_ref[...].astype(jnp.bfloat16),
                 preferred_element_type=jnp.float32)
    h = (hg * hu).astype(jnp.bfloat16)
    ysh = jnp.dot(h, wd_ref[...].astype(jnp.bfloat16),
                  preferred_element_type=jnp.float32) * gfix
    out_ref[...] = y0_ref[...] * w0 + y1_ref[...] * w1 + ysh


def _combine_call(ygath, x, wg_sh, wu_sh, wd_sh, wv, half, T, prev=None):
    TK, D = ygath.shape
    TH = TK // TOP_K
    BT = 512
    nb = TH // BT
    off = half * nb
    in_specs = [
        pl.BlockSpec((BT, D), lambda m: (m, 0)),
        pl.BlockSpec((BT, D), lambda m, _nb=nb: (m + _nb, 0)),
        pl.BlockSpec((BT, D), lambda m, _o=off: (m + _o, 0)),
        pl.BlockSpec(wg_sh.shape, lambda m: (0, 0)),
        pl.BlockSpec(wu_sh.shape, lambda m: (0, 0)),
        pl.BlockSpec(wd_sh.shape, lambda m: (0, 0)),
        pl.BlockSpec((BT, 128), lambda m, _o=off: (m + _o, 0)),
    ]
    args = [ygath, ygath, x, wg_sh, wu_sh, wd_sh, wv]
    aliases = {}
    if prev is not None:
        in_specs.append(pl.BlockSpec(memory_space=pl.ANY))
        args.append(prev)
        aliases = {7: 0}
    return pl.pallas_call(
        _combine_body,
        grid=(nb,),
        in_specs=in_specs,
        out_specs=pl.BlockSpec((BT, D), lambda m, _o=off: (m + _o, 0)),
        out_shape=jax.ShapeDtypeStruct((T, D), jnp.float32),
        input_output_aliases=aliases,
        compiler_params=pltpu.CompilerParams(
            dimension_semantics=("arbitrary",)),
    )(*args)


@jax.jit
def _run(hidden_states, W_router, Wg_dyn, Wu_dyn, Wd_dyn, Wg_sh, Wu_sh, Wd_sh):
    B, S, D = hidden_states.shape
    x = hidden_states.reshape(-1, D)
    T = x.shape[0]
    NSLOT = TOP_K * T
    P = NSLOT + E_DYN * BM

    wr_pad = jnp.zeros((D, 128), jnp.float32).at[:, :E_DYN + 1].set(W_router)
    wv, pp, te2 = _route_call(x, wr_pad)

    posp = jnp.concatenate([pp[:, 0], pp[:, 1]])
    tile_e = te2[:P // BM, 0]
    tile_v = te2[:P // BM, 1]

    xp = _sc_scatter(x, posp.reshape(32, -1, 16), P)
    yp = _group_call(tile_e, tile_v, xp, Wg_dyn, Wu_dyn, Wd_dyn)
    ygath = _sc_gather(yp, posp.reshape(32, -1, 16))
    out = _combine_call(ygath, x, Wg_sh[0], Wu_sh[0], Wd_sh[0], wv, 0, T)
    return out.reshape(B, S, D)


def _sc_gather(table, idx3):
    fn = _make_sc_gather(table.shape[0], table.shape[1],
                         idx3.shape[0], idx3.shape[1], idx3.shape[2], 2,
                         table.dtype.type)
    return fn(table, idx3)


def _sc_scatter(src, idx3, P):
    fn = _make_sc_scatter(src.shape[0], src.shape[1], P,
                          idx3.shape[0], idx3.shape[1], idx3.shape[2], 2)
    return fn(src, idx3)


def kernel(hidden_states, W_router, Wg_dyn, Wu_dyn, Wd_dyn, Wg_sh, Wu_sh, Wd_sh):
    return _run(hidden_states, W_router, Wg_dyn, Wu_dyn, Wd_dyn,
                Wg_sh, Wu_sh, Wd_sh)
